# baseline (speedup 1.0000x reference)
import functools

import jax
import jax.numpy as jnp
from jax import lax
from jax.experimental import pallas as pl
from jax.experimental.pallas import tpu as pltpu
from jax.experimental.pallas import tpu_sc as plsc

_B, _C, _H, _W = 8, 256, 64, 64
_Ho, _Wo = 128, 128
_NW = 32
_RPW = (_B * _H) // _NW
_CB = _C // 2
_JH = _W // 2
_NCHB = _CB // 16


def _unpool_sc(xt, it):
    mesh = plsc.VectorSubcoreMesh(core_axis_name="c", subcore_axis_name="s")

    @functools.partial(
        pl.kernel,
        mesh=mesh,
        compiler_params=pltpu.CompilerParams(needs_layout_passes=False),
        out_type=jax.ShapeDtypeStruct((_B, _C, _Ho, _Wo), jnp.float32),
        scratch_types=[
            pltpu.VMEM((_JH, _C), jnp.float32),
            pltpu.VMEM((_JH, _C), jnp.float32),
            pltpu.VMEM((_JH, _C), jnp.float32),
            pltpu.VMEM((_W, _C), jnp.int32),
            pltpu.VMEM((_W, _C), jnp.int32),
            pltpu.VMEM((_CB, 2, _Wo), jnp.float32),
            pltpu.VMEM((_CB, 2, _Wo), jnp.float32),
            pltpu.SemaphoreType.DMA,
            pltpu.SemaphoreType.DMA,
            pltpu.SemaphoreType.DMA,
            pltpu.SemaphoreType.DMA,
            pltpu.SemaphoreType.DMA,
            pltpu.SemaphoreType.DMA,
            pltpu.SemaphoreType.DMA,
        ],
    )
    def body(x_hbm, i_hbm, o_hbm, x_v0, x_v1, x_v2, i_v0, i_v1, st0, st1,
             smx0, smx1, smx2, smi0, smi1, smf0, smf1):
        w = lax.axis_index("s") * 2 + lax.axis_index("c")
        r0 = w * _RPW
        zf = jnp.zeros((16,), jnp.float32)
        iota = lax.iota(jnp.int32, 16)
        xbufs = [x_v0, x_v1, x_v2]
        ibufs = [i_v0, i_v1]
        stbufs = [st0, st1]
        smx, smi, smf = [smx0, smx1, smx2], [smi0, smi1], [smf0, smf1]

        def row_coords(t):
            rid = r0 + t
            return rid >> 6, rid & 63

        def issue_xhalf(t, jh, xb):
            bt, i_img = row_coords(lax.min(t, _RPW - 1))
            pltpu.async_copy(
                x_hbm.at[bt, i_img, pl.ds(jh * _JH, _JH)], xbufs[xb], smx[xb])

        def wait_xhalf(xb):
            pltpu.make_async_copy(x_hbm.at[0, 0, pl.ds(0, _JH)],
                                  xbufs[xb], smx[xb]).wait()

        def issue_irow(t, ib):
            bt, i_img = row_coords(lax.min(t, _RPW - 1))
            pltpu.async_copy(i_hbm.at[bt, i_img], ibufs[ib], smi[ib])

        def wait_irow(ib):
            pltpu.make_async_copy(i_hbm.at[0, 0], ibufs[ib], smi[ib]).wait()

        def out_slice(t, h):
            bt, i_img = row_coords(t)
            return o_hbm.at[bt, pl.ds(h * _CB, _CB), pl.ds(2 * i_img, 2),
                            pl.ds(0, _Wo)]

        def flush(h, t):
            pltpu.async_copy(stbufs[h], out_slice(t, h), smf[h])

        def flush_wait(h, t):
            pltpu.make_async_copy(stbufs[h], out_slice(t, h), smf[h]).wait()

        def scat(t, h, xa, xb_, ib):
            _, i_img = row_coords(t)
            ir, st = ibufs[ib], stbufs[h]
            ri2 = 2 * i_img

            def make_kbody(xr, joff):
                def kbody(j, c):
                    ivs, xvs = [], []
                    for u in range(_NCHB):
                        col = h * _CB + u * 16
                        ivs.append(ir[j + joff, pl.ds(col, 16)])
                        if xr is not None:
                            xvs.append(xr[j, pl.ds(col, 16)])
                    for u in range(_NCHB):
                        cvec = iota + u * 16
                        drv = (ivs[u] >> 7) - ri2
                        dcv = ivs[u] & 127
                        val = zf if xr is None else xvs[u]
                        plsc.store_scatter(st, [cvec, drv, dcv], val)
                    return c
                return kbody

            lax.fori_loop(0, _JH, make_kbody(xa, 0), 0)
            lax.fori_loop(0, _JH, make_kbody(xb_, _JH), 0)

        issue_xhalf(0, 0, 0)
        issue_xhalf(0, 1, 1)
        issue_irow(0, 0)

        def zero_body(k, carry):
            st0[k >> 4, (k >> 3) & 1, pl.ds((k & 7) * 16, 16)] = zf
            st1[k >> 4, (k >> 3) & 1, pl.ds((k & 7) * 16, 16)] = zf
            return carry

        lax.fori_loop(0, (_CB * 2 * _Wo) // 16, zero_body, 0)

        def run_row(t, tmod6, first):
            tA = (2 * tmod6) % 3
            tB = (2 * tmod6 + 1) % 3
            tN = (2 * tmod6 + 2) % 3
            ibc = tmod6 % 2
            ibp = (tmod6 + 1) % 2
            wait_xhalf(tA)
            wait_xhalf(tB)
            wait_irow(ibc)
            issue_xhalf(t + 1, 0, tN)
            if not first:
                flush_wait(0, t - 1)
                scat(t - 1, 0, None, None, ibp)
            scat(t, 0, xbufs[tA], xbufs[tB], ibc)
            flush(0, t)
            if not first:
                flush_wait(1, t - 1)
                scat(t - 1, 1, None, None, ibp)
            issue_irow(t + 1, ibp)
            scat(t, 1, xbufs[tA], xbufs[tB], ibc)
            flush(1, t)
            issue_xhalf(t + 1, 1, tA)

        for t in range(4):
            run_row(t, t, first=(t == 0))

        def steady(g, carry):
            for e in range(6):
                run_row(4 + 6 * g + e, 4 + e, first=False)
            return carry

        lax.fori_loop(0, 2, steady, 0)

        flush_wait(0, _RPW - 1)
        flush_wait(1, _RPW - 1)
        wait_xhalf(2)
        wait_xhalf(0)
        wait_irow(0)

    return body(xt, it)


def kernel(x, indices, output_size):
    del output_size
    xt = x.transpose(0, 2, 3, 1)
    it = indices.transpose(0, 2, 3, 1)
    return _unpool_sc(xt, it)


# speedup vs baseline: 1.1258x; 1.1258x over previous
import functools

import jax
import jax.numpy as jnp
from jax import lax
from jax.experimental import pallas as pl
from jax.experimental.pallas import tpu as pltpu
from jax.experimental.pallas import tpu_sc as plsc

_B, _C, _H, _W = 8, 256, 64, 64
_Ho, _Wo = 128, 128
_NW = 32
_RPW = (_B * _H) // _NW
_NBLK = _RPW * 2
_CB = _C // 2
_NCHB = _CB // 16


def _unpool_sc(xt, it):
    mesh = plsc.VectorSubcoreMesh(core_axis_name="c", subcore_axis_name="s")

    @functools.partial(
        pl.kernel,
        mesh=mesh,
        compiler_params=pltpu.CompilerParams(needs_layout_passes=False),
        out_type=jax.ShapeDtypeStruct((_B, _C, _Ho, _Wo), jnp.float32),
        scratch_types=[
            pltpu.VMEM((_W, _CB), jnp.float32),
            pltpu.VMEM((_W, _CB), jnp.float32),
            pltpu.VMEM((_W, _CB), jnp.int32),
            pltpu.VMEM((_W, _CB), jnp.int32),
            pltpu.VMEM((_W, _CB), jnp.int32),
            pltpu.VMEM((_W, _CB), jnp.int32),
            pltpu.VMEM((_CB, 2, _Wo), jnp.float32),
            pltpu.VMEM((_CB, 2, _Wo), jnp.float32),
            pltpu.SemaphoreType.DMA,
            pltpu.SemaphoreType.DMA,
            pltpu.SemaphoreType.DMA,
            pltpu.SemaphoreType.DMA,
            pltpu.SemaphoreType.DMA,
            pltpu.SemaphoreType.DMA,
            pltpu.SemaphoreType.DMA,
            pltpu.SemaphoreType.DMA,
        ],
    )
    def body(x_hbm, i_hbm, o_hbm, x_v0, x_v1, i_v0, i_v1, i_v2, i_v3,
             st0, st1, smx0, smx1, smi0, smi1, smi2, smi3, smf0, smf1):
        w = lax.axis_index("s") * 2 + lax.axis_index("c")
        r0 = w * _RPW
        zf = jnp.zeros((16,), jnp.float32)
        iota = lax.iota(jnp.int32, 16)
        xbufs = [x_v0, x_v1]
        ibufs = [i_v0, i_v1, i_v2, i_v3]
        stbufs = [st0, st1]
        smx, smi, smf = [smx0, smx1], [smi0, smi1, smi2, smi3], [smf0, smf1]

        def blk_coords(q):
            rid = r0 + (q >> 1)
            return rid >> 6, rid & 63

        def issue_in(q, h, xb, ib):
            bq, iq = blk_coords(q)
            c0 = h * _CB
            pltpu.async_copy(x_hbm.at[bq, iq, pl.ds(0, _W), pl.ds(c0, _CB)],
                             xbufs[xb], smx[xb])
            pltpu.async_copy(i_hbm.at[bq, iq, pl.ds(0, _W), pl.ds(c0, _CB)],
                             ibufs[ib], smi[ib])

        def wait_in(xb, ib):
            pltpu.make_async_copy(
                x_hbm.at[0, 0, pl.ds(0, _W), pl.ds(0, _CB)],
                xbufs[xb], smx[xb]).wait()
            pltpu.make_async_copy(
                i_hbm.at[0, 0, pl.ds(0, _W), pl.ds(0, _CB)],
                ibufs[ib], smi[ib]).wait()

        def out_slice(q, h):
            bq, iq = blk_coords(q)
            return o_hbm.at[bq, pl.ds(h * _CB, _CB), pl.ds(2 * iq, 2),
                            pl.ds(0, _Wo)]

        def flush(sb, q, h):
            pltpu.async_copy(stbufs[sb], out_slice(q, h), smf[sb])

        def flush_wait(sb, q, h):
            pltpu.make_async_copy(stbufs[sb], out_slice(q, h), smf[sb]).wait()

        def scat_like(q, ib, sb, xb):
            _, iq = blk_coords(q)
            ir, st = ibufs[ib], stbufs[sb]
            xr = None if xb is None else xbufs[xb]
            ri2 = 2 * iq

            def kbody(j, c):
                ivs, xvs = [], []
                for u in range(_NCHB):
                    ivs.append(ir[j, pl.ds(u * 16, 16)])
                    if xr is not None:
                        xvs.append(xr[j, pl.ds(u * 16, 16)])
                for u in range(_NCHB):
                    cvec = iota + u * 16
                    drv = (ivs[u] >> 7) - ri2
                    dcv = ivs[u] & 127
                    val = zf if xr is None else xvs[u]
                    plsc.store_scatter(st, [cvec, drv, dcv], val)
                return c

            lax.fori_loop(0, _W, kbody, 0)

        issue_in(0, 0, 0, 0)

        def zero_body(k, carry):
            st0[k >> 4, (k >> 3) & 1, pl.ds((k & 7) * 16, 16)] = zf
            st1[k >> 4, (k >> 3) & 1, pl.ds((k & 7) * 16, 16)] = zf
            return carry

        lax.fori_loop(0, (_CB * 2 * _Wo) // 16, zero_body, 0)

        def run_block(q, s, peeled):
            h = s & 1
            xb, ib, sb = s % 2, s % 4, s % 2
            wait_in(xb, ib)
            qn = lax.min(q + 1, _NBLK - 1)
            issue_in(qn, (s + 1) & 1, (s + 1) % 2, (s + 1) % 4)
            if not (peeled and s < 2):
                flush_wait(sb, q - 2, h)
                scat_like(q - 2, (s + 2) % 4, sb, None)
            scat_like(q, ib, sb, xb)
            flush(sb, q, h)

        for s in range(4):
            run_block(s, s, peeled=True)

        def steady(mi, carry):
            for s in range(4):
                run_block(4 * mi + s, s, peeled=False)
            return carry

        lax.fori_loop(1, _NBLK // 4, steady, 0)

        flush_wait(0, _NBLK - 2, 0)
        flush_wait(1, _NBLK - 1, 1)
        pltpu.make_async_copy(x_hbm.at[0, 0, pl.ds(0, _W), pl.ds(0, _CB)],
                              xbufs[0], smx[0]).wait()
        pltpu.make_async_copy(i_hbm.at[0, 0, pl.ds(0, _W), pl.ds(0, _CB)],
                              ibufs[0], smi[0]).wait()

    return body(xt, it)


def kernel(x, indices, output_size):
    del output_size
    xt = x.transpose(0, 2, 3, 1)
    it = indices.transpose(0, 2, 3, 1)
    return _unpool_sc(xt, it)
